# baseline (speedup 1.0000x reference)
import jax
import jax.numpy as jnp
from jax import lax
from jax.experimental import pallas as pl
from jax.experimental.pallas import tpu as pltpu
from jax.experimental.pallas import tpu_sc as plsc

N = 100000
DIM = 32
ROWS_PAD = 1007616
NW = 32

TCOLS = 8192
QUART = TCOLS // 4
QUART_LOG2 = QUART.bit_length() - 1
OUT_BLK = TCOLS * DIM // 128
SCRATCH_ROWS = ROWS_PAD * DIM // 128
GRID = -(-ROWS_PAD // TCOLS)

OCOLS = 8192
OQ = OCOLS // 4
N_PAD = 106496
B_PER_W = N_PAD // NW
OGRID = N_PAD // OCOLS
N_MINOR = 100096

EA_CH = 12800
EA_NCH = 125


def _ea_chunk_copy(ea_ref, ea_out, sem, c):
    return pltpu.make_async_copy(
        ea_ref.at[:, pl.ds(c * EA_CH, EA_CH)],
        ea_out.at[:, pl.ds(c * EA_CH, EA_CH)], sem)


def _transpose_body(x_ref, ea_ref, y_ref, ea_out, sem_ea):
    t = pl.program_id(0)

    @pl.when(t < EA_NCH - 2)
    def _start():
        _ea_chunk_copy(ea_ref, ea_out, sem_ea, t).start()

    @pl.when(t == 0)
    def _start_tail():
        _ea_chunk_copy(ea_ref, ea_out, sem_ea, EA_NCH - 2).start()
        _ea_chunk_copy(ea_ref, ea_out, sem_ea, EA_NCH - 1).start()

    x = x_ref[...]
    x4 = jnp.concatenate(
        [x[:, 0:QUART], x[:, QUART:2 * QUART], x[:, 2 * QUART:3 * QUART],
         x[:, 3 * QUART:4 * QUART]], axis=0)
    y_ref[...] = jnp.transpose(x4, (1, 0))

    @pl.when(t == GRID - 1)
    def _wait():
        for c in range(EA_NCH):
            _ea_chunk_copy(ea_ref, ea_out, sem_ea, c).wait()


def _detile(table_t, edge_attr_t):
    return pl.pallas_call(
        _transpose_body,
        grid=(GRID,),
        in_specs=[
            pl.BlockSpec((DIM, TCOLS), lambda t: (0, t)),
            pl.BlockSpec(memory_space=pltpu.HBM),
        ],
        out_specs=[
            pl.BlockSpec((OUT_BLK, 128), lambda t: (t, 0)),
            pl.BlockSpec(memory_space=pltpu.HBM),
        ],
        out_shape=[
            jax.ShapeDtypeStruct((SCRATCH_ROWS, 128), jnp.float32),
            jax.ShapeDtypeStruct(edge_attr_t.shape, edge_attr_t.dtype),
        ],
        scratch_shapes=[pltpu.SemaphoreType.DMA],
    )(table_t, edge_attr_t)


def _back_body(g_ref, o_ref):
    z = jnp.transpose(g_ref[...], (1, 0))
    o_ref[...] = jnp.concatenate(
        [z[0:DIM], z[DIM:2 * DIM], z[2 * DIM:3 * DIM], z[3 * DIM:4 * DIM]],
        axis=1)


def _back_transpose(g_flat):
    return pl.pallas_call(
        _back_body,
        grid=(OGRID,),
        in_specs=[pl.BlockSpec((OQ, 128), lambda t: (t, 0))],
        out_specs=pl.BlockSpec((DIM, OCOLS), lambda t: (0, t)),
        out_shape=jax.ShapeDtypeStruct((DIM, N_MINOR), jnp.float32),
    )(g_flat)


def _gather_body(table_hbm, idx_hbm, out_hbm, idx_v, rows_v, sem):
    wid = lax.axis_index("s") * 2 + lax.axis_index("c")
    base = wid * B_PER_W
    pltpu.sync_copy(idx_hbm.at[pl.ds(base, B_PER_W)], idx_v)
    pltpu.async_copy(table_hbm.at[idx_v], rows_v, sem).wait()
    pltpu.sync_copy(rows_v, out_hbm.at[pl.ds(base, B_PER_W)])


def _gather(table_rows, idx_pad):
    mesh = plsc.VectorSubcoreMesh(core_axis_name="c", subcore_axis_name="s")
    f = pl.kernel(
        _gather_body,
        out_type=jax.ShapeDtypeStruct((N_PAD, DIM), jnp.float32),
        mesh=mesh,
        scratch_types=[
            pltpu.VMEM((B_PER_W,), jnp.int32),
            pltpu.VMEM((B_PER_W, DIM), jnp.float32),
            pltpu.SemaphoreType.DMA,
        ],
        compiler_params=pltpu.CompilerParams(use_tc_tiling_on_sc=False),
    )
    return f(table_rows, idx_pad)


def kernel(x, edge_index, edge_attr, batch, depth, ptr, table):
    table_t = table.T
    scratch, ea_out_t = _detile(table_t, edge_attr.T)
    table_rows = scratch.reshape(ROWS_PAD, DIM)
    idx = x.reshape(-1)
    u = idx & (TCOLS - 1)
    gidx = (idx - u) + ((u & (QUART - 1)) << 2) + (u >> QUART_LOG2)
    gp = jnp.pad(gidx, (0, N_PAD - N))
    gidx2 = gp.reshape(OGRID, 4, OQ).transpose(0, 2, 1).reshape(-1)
    g_rows = _gather(table_rows, gidx2)
    out_t = _back_transpose(g_rows.reshape(N_PAD * DIM // 128, 128))
    return (out_t.T[:N], edge_index, ea_out_t.T, batch, depth, ptr)


# speedup vs baseline: 9.2703x; 9.2703x over previous
import jax
import jax.numpy as jnp
from jax import lax
from jax.experimental import pallas as pl
from jax.experimental.pallas import tpu as pltpu
from jax.experimental.pallas import tpu_sc as plsc

N = 100000
DIM = 32
ROWS_PAD = 1007616
NW = 32

TCOLS = 8192
QUART = TCOLS // 4
QUART_LOG2 = QUART.bit_length() - 1
OUT_BLK = TCOLS * DIM // 128
SCRATCH_ROWS = ROWS_PAD * DIM // 128
GRID = -(-ROWS_PAD // TCOLS)

OCOLS = 8192
OQ = OCOLS // 4
N_PAD = 106496
B_PER_W = N_PAD // NW
OGRID = N_PAD // OCOLS
N_MINOR = 100096


def _transpose_body(x_ref, y_ref):
    x = x_ref[...]
    x4 = jnp.concatenate(
        [x[:, 0:QUART], x[:, QUART:2 * QUART], x[:, 2 * QUART:3 * QUART],
         x[:, 3 * QUART:4 * QUART]], axis=0)
    y_ref[...] = jnp.transpose(x4, (1, 0))


def _detile(table_t):
    return pl.pallas_call(
        _transpose_body,
        grid=(GRID,),
        in_specs=[pl.BlockSpec((DIM, TCOLS), lambda t: (0, t))],
        out_specs=pl.BlockSpec((OUT_BLK, 128), lambda t: (t, 0)),
        out_shape=jax.ShapeDtypeStruct((SCRATCH_ROWS, 128), jnp.float32),
    )(table_t)


def _back_body(g_ref, o_ref):
    z = jnp.transpose(g_ref[...], (1, 0))
    o_ref[...] = jnp.concatenate(
        [z[0:DIM], z[DIM:2 * DIM], z[2 * DIM:3 * DIM], z[3 * DIM:4 * DIM]],
        axis=1)


def _back_transpose(g_flat):
    return pl.pallas_call(
        _back_body,
        grid=(OGRID,),
        in_specs=[pl.BlockSpec((OQ, 128), lambda t: (t, 0))],
        out_specs=pl.BlockSpec((DIM, OCOLS), lambda t: (0, t)),
        out_shape=jax.ShapeDtypeStruct((DIM, N_MINOR), jnp.float32),
    )(g_flat)


def _gather_body(table_hbm, idx_hbm, out_hbm, idx_v, rows_v, sem):
    wid = lax.axis_index("s") * 2 + lax.axis_index("c")
    base = wid * B_PER_W
    pltpu.sync_copy(idx_hbm.at[pl.ds(base, B_PER_W)], idx_v)
    pltpu.async_copy(table_hbm.at[idx_v], rows_v, sem).wait()
    pltpu.sync_copy(rows_v, out_hbm.at[pl.ds(base, B_PER_W)])


def _gather(table_rows, idx_pad):
    mesh = plsc.VectorSubcoreMesh(core_axis_name="c", subcore_axis_name="s")
    f = pl.kernel(
        _gather_body,
        out_type=jax.ShapeDtypeStruct((N_PAD, DIM), jnp.float32),
        mesh=mesh,
        scratch_types=[
            pltpu.VMEM((B_PER_W,), jnp.int32),
            pltpu.VMEM((B_PER_W, DIM), jnp.float32),
            pltpu.SemaphoreType.DMA,
        ],
        compiler_params=pltpu.CompilerParams(use_tc_tiling_on_sc=False),
    )
    return f(table_rows, idx_pad)


def kernel(x, edge_index, edge_attr, batch, depth, ptr, table):
    table_t = table.T
    scratch = _detile(table_t)
    table_rows = scratch.reshape(ROWS_PAD, DIM)
    idx = x.reshape(-1)
    u = idx & (TCOLS - 1)
    gidx = (idx - u) + ((u & (QUART - 1)) << 2) + (u >> QUART_LOG2)
    gp = jnp.pad(gidx, (0, N_PAD - N))
    gidx2 = gp.reshape(OGRID, 4, OQ).transpose(0, 2, 1).reshape(-1)
    g_rows = _gather(table_rows, gidx2)
    out_t = _back_transpose(g_rows.reshape(N_PAD * DIM // 128, 128))
    return (out_t.T[:N], edge_index, edge_attr, batch, depth, ptr)


# speedup vs baseline: 11.3412x; 1.2234x over previous
import jax
import jax.numpy as jnp
from jax import lax
from jax.experimental import pallas as pl
from jax.experimental.pallas import tpu as pltpu
from jax.experimental.pallas import tpu_sc as plsc

N = 100000
DIM = 32
ROWS_PAD = 1007616
NW = 32

TCOLS = 8192
QUART = TCOLS // 4
QUART_LOG2 = QUART.bit_length() - 1
OUT_BLK = TCOLS * DIM // 128
SCRATCH_ROWS = ROWS_PAD * DIM // 128
GRID = -(-ROWS_PAD // TCOLS)

OCOLS = 2048
OQ = OCOLS // 4
N_PAD = 100352
B_PER_W = N_PAD // NW
OGRID = N_PAD // OCOLS
N_MINOR = 100096


def _transpose_body(x_ref, y_ref):
    x = x_ref[...]
    x4 = jnp.concatenate(
        [x[:, 0:QUART], x[:, QUART:2 * QUART], x[:, 2 * QUART:3 * QUART],
         x[:, 3 * QUART:4 * QUART]], axis=0)
    y_ref[...] = jnp.transpose(x4, (1, 0))


def _detile(table_t):
    return pl.pallas_call(
        _transpose_body,
        grid=(GRID,),
        in_specs=[pl.BlockSpec((DIM, TCOLS), lambda t: (0, t))],
        out_specs=pl.BlockSpec((OUT_BLK, 128), lambda t: (t, 0)),
        out_shape=jax.ShapeDtypeStruct((SCRATCH_ROWS, 128), jnp.float32),
    )(table_t)


def _back_body(g_ref, o_ref):
    z = jnp.transpose(g_ref[...], (1, 0))
    o_ref[...] = jnp.concatenate(
        [z[0:DIM], z[DIM:2 * DIM], z[2 * DIM:3 * DIM], z[3 * DIM:4 * DIM]],
        axis=1)


def _back_transpose(g_flat):
    return pl.pallas_call(
        _back_body,
        grid=(OGRID,),
        in_specs=[pl.BlockSpec((OQ, 128), lambda t: (t, 0))],
        out_specs=pl.BlockSpec((DIM, OCOLS), lambda t: (0, t)),
        out_shape=jax.ShapeDtypeStruct((DIM, N_MINOR), jnp.float32),
    )(g_flat)


EA_CH = 6400
EA_NCH = 250


def _ea_copy_body(src_hbm, dst_hbm, buf_v):
    wid = lax.axis_index("s") * 2 + lax.axis_index("c")
    for k in range(8):
        cid = wid * 8 + k

        @pl.when(cid < EA_NCH)
        def _():
            off = cid * EA_CH
            pltpu.sync_copy(src_hbm.at[:, pl.ds(off, EA_CH)], buf_v)
            pltpu.sync_copy(buf_v, dst_hbm.at[:, pl.ds(off, EA_CH)])


def _sc_ea_copy(ea_t):
    mesh = plsc.VectorSubcoreMesh(core_axis_name="c", subcore_axis_name="s")
    f = pl.kernel(
        _ea_copy_body,
        out_type=jax.ShapeDtypeStruct(ea_t.shape, ea_t.dtype),
        mesh=mesh,
        scratch_types=[pltpu.VMEM((16, EA_CH), jnp.float32)],
        compiler_params=pltpu.CompilerParams(use_tc_tiling_on_sc=True),
    )
    return f(ea_t)


def _gather_body(table_hbm, idx_hbm, out_hbm, idx_v, rows_v, sem):
    wid = lax.axis_index("s") * 2 + lax.axis_index("c")
    base = wid * B_PER_W
    pltpu.sync_copy(idx_hbm.at[pl.ds(base, B_PER_W)], idx_v)
    pltpu.async_copy(table_hbm.at[idx_v], rows_v, sem).wait()
    pltpu.sync_copy(rows_v, out_hbm.at[pl.ds(base, B_PER_W)])


def _gather(table_rows, idx_pad):
    mesh = plsc.VectorSubcoreMesh(core_axis_name="c", subcore_axis_name="s")
    f = pl.kernel(
        _gather_body,
        out_type=jax.ShapeDtypeStruct((N_PAD, DIM), jnp.float32),
        mesh=mesh,
        scratch_types=[
            pltpu.VMEM((B_PER_W,), jnp.int32),
            pltpu.VMEM((B_PER_W, DIM), jnp.float32),
            pltpu.SemaphoreType.DMA,
        ],
        compiler_params=pltpu.CompilerParams(use_tc_tiling_on_sc=False),
    )
    return f(table_rows, idx_pad)


def kernel(x, edge_index, edge_attr, batch, depth, ptr, table):
    table_t = table.T
    scratch = _detile(table_t)
    table_rows = scratch.reshape(ROWS_PAD, DIM)
    idx = x.reshape(-1)
    u = idx & (TCOLS - 1)
    gidx = (idx - u) + ((u & (QUART - 1)) << 2) + (u >> QUART_LOG2)
    gp = jnp.pad(gidx, (0, N_PAD - N))
    gidx2 = gp.reshape(OGRID, 4, OQ).transpose(0, 2, 1).reshape(-1)
    g_rows = _gather(table_rows, gidx2)
    out_t = _back_transpose(g_rows.reshape(N_PAD * DIM // 128, 128))
    ea_out_t = _sc_ea_copy(edge_attr.T)
    return (out_t.T[:N], edge_index, ea_out_t.T, batch, depth, ptr)


# speedup vs baseline: 11.9235x; 1.0513x over previous
import jax
import jax.numpy as jnp
from jax import lax
from jax.experimental import pallas as pl
from jax.experimental.pallas import tpu as pltpu
from jax.experimental.pallas import tpu_sc as plsc

N = 100000
DIM = 32
ROWS_PAD = 1007616
NW = 32

TCOLS = 8192
QUART = TCOLS // 4
QUART_LOG2 = QUART.bit_length() - 1
OUT_BLK = TCOLS * DIM // 128
SCRATCH_ROWS = ROWS_PAD * DIM // 128
GRID = -(-ROWS_PAD // TCOLS)

OCOLS = 2048
OQ = OCOLS // 4
N_PAD = 100352
B_PER_W = N_PAD // NW
OGRID = N_PAD // OCOLS
N_MINOR = 100096


def _transpose_body(x_ref, y_ref):
    x = x_ref[...]
    x4 = jnp.concatenate(
        [x[:, 0:QUART], x[:, QUART:2 * QUART], x[:, 2 * QUART:3 * QUART],
         x[:, 3 * QUART:4 * QUART]], axis=0)
    y_ref[...] = jnp.transpose(x4, (1, 0))


def _detile(table_t):
    return pl.pallas_call(
        _transpose_body,
        grid=(GRID,),
        in_specs=[pl.BlockSpec((DIM, TCOLS), lambda t: (0, t))],
        out_specs=pl.BlockSpec((OUT_BLK, 128), lambda t: (t, 0)),
        out_shape=jax.ShapeDtypeStruct((SCRATCH_ROWS, 128), jnp.float32),
    )(table_t)


def _back_body(g_ref, o_ref):
    z = jnp.transpose(g_ref[...], (1, 0))
    o_ref[...] = jnp.concatenate(
        [z[0:DIM], z[DIM:2 * DIM], z[2 * DIM:3 * DIM], z[3 * DIM:4 * DIM]],
        axis=1)


def _back_transpose(g_flat):
    return pl.pallas_call(
        _back_body,
        grid=(OGRID,),
        in_specs=[pl.BlockSpec((OQ, 128), lambda t: (t, 0))],
        out_specs=pl.BlockSpec((DIM, OCOLS), lambda t: (0, t)),
        out_shape=jax.ShapeDtypeStruct((DIM, N_MINOR), jnp.float32),
    )(g_flat)


EA_CH = 6400
EA_NCH = 250


def _ea_copy_body(src_hbm, dst_hbm, buf_v):
    wid = lax.axis_index("s") * 2 + lax.axis_index("c")
    for k in range(8):
        cid = wid * 8 + k

        @pl.when(cid < EA_NCH)
        def _():
            off = cid * EA_CH
            pltpu.sync_copy(src_hbm.at[:, pl.ds(off, EA_CH)], buf_v)
            pltpu.sync_copy(buf_v, dst_hbm.at[:, pl.ds(off, EA_CH)])


def _sc_ea_copy(ea_t):
    mesh = plsc.VectorSubcoreMesh(core_axis_name="c", subcore_axis_name="s")
    f = pl.kernel(
        _ea_copy_body,
        out_type=jax.ShapeDtypeStruct(ea_t.shape, ea_t.dtype),
        mesh=mesh,
        scratch_types=[pltpu.VMEM((16, EA_CH), jnp.float32)],
        compiler_params=pltpu.CompilerParams(use_tc_tiling_on_sc=True),
    )
    return f(ea_t)


N_PAD_X = 102400
XSL = 3 * OCOLS


def _gather_body(table_hbm, x_hbm, out_hbm, xv, idx_v, rows_v, sem):
    wid = lax.axis_index("s") * 2 + lax.axis_index("c")
    base = wid * B_PER_W
    t0 = base // OCOLS
    pltpu.sync_copy(x_hbm.at[pl.ds(t0 * OCOLS, XSL)], xv)
    lane = lax.iota(jnp.int32, 16)
    for i in range(B_PER_W // 16):
        g = base + 16 * i + lane
        r = g & (OCOLS - 1)
        j_local = (g - r) - t0 * OCOLS + ((r & 3) << 9) + (r >> 2)
        xval = plsc.load_gather(xv, [j_local])
        u = xval & (TCOLS - 1)
        gv = (xval - u) + ((u & (QUART - 1)) << 2) + (u >> QUART_LOG2)
        idx_v[pl.ds(16 * i, 16)] = gv
    pltpu.async_copy(table_hbm.at[idx_v], rows_v, sem).wait()
    pltpu.sync_copy(rows_v, out_hbm.at[pl.ds(base, B_PER_W)])


def _gather(table_rows, x_pad):
    mesh = plsc.VectorSubcoreMesh(core_axis_name="c", subcore_axis_name="s")
    f = pl.kernel(
        _gather_body,
        out_type=jax.ShapeDtypeStruct((N_PAD, DIM), jnp.float32),
        mesh=mesh,
        scratch_types=[
            pltpu.VMEM((XSL,), jnp.int32),
            pltpu.VMEM((B_PER_W,), jnp.int32),
            pltpu.VMEM((B_PER_W, DIM), jnp.float32),
            pltpu.SemaphoreType.DMA,
        ],
        compiler_params=pltpu.CompilerParams(use_tc_tiling_on_sc=False,
                                             needs_layout_passes=False),
    )
    return f(table_rows, x_pad)


def kernel(x, edge_index, edge_attr, batch, depth, ptr, table):
    ea_out_t = _sc_ea_copy(edge_attr.T)
    table_t = table.T
    scratch = _detile(table_t)
    table_rows = scratch.reshape(ROWS_PAD, DIM)
    x_pad = jnp.pad(x.reshape(-1), (0, N_PAD_X - N))
    g_rows = _gather(table_rows, x_pad)
    out_t = _back_transpose(g_rows.reshape(N_PAD * DIM // 128, 128))
    return (out_t.T[:N], edge_index, ea_out_t.T, batch, depth, ptr)


# speedup vs baseline: 12.2523x; 1.0276x over previous
import jax
import jax.numpy as jnp
from jax import lax
from jax.experimental import pallas as pl
from jax.experimental.pallas import tpu as pltpu
from jax.experimental.pallas import tpu_sc as plsc

N = 100000
DIM = 32
ROWS_PAD = 1007616
NW = 32

TCOLS = 8192
QUART = TCOLS // 4
QUART_LOG2 = QUART.bit_length() - 1
OUT_BLK = TCOLS * DIM // 128
SCRATCH_ROWS = ROWS_PAD * DIM // 128
GRID = -(-ROWS_PAD // TCOLS)

OCOLS = 2048
OQ = OCOLS // 4
N_PAD = 100352
B_PER_W = N_PAD // NW
OGRID = N_PAD // OCOLS
N_MINOR = 100096


def _transpose_body(x_ref, y_ref):
    x = x_ref[...]
    x4 = jnp.concatenate(
        [x[:, 0:QUART], x[:, QUART:2 * QUART], x[:, 2 * QUART:3 * QUART],
         x[:, 3 * QUART:4 * QUART]], axis=0)
    y_ref[...] = jnp.transpose(x4, (1, 0))


def _detile(table_t):
    return pl.pallas_call(
        _transpose_body,
        grid=(GRID,),
        in_specs=[pl.BlockSpec((DIM, TCOLS), lambda t: (0, t))],
        out_specs=pl.BlockSpec((OUT_BLK, 128), lambda t: (t, 0)),
        out_shape=jax.ShapeDtypeStruct((SCRATCH_ROWS, 128), jnp.float32),
    )(table_t)


def _back_body(g_ref, o_ref):
    z = jnp.transpose(g_ref[...], (1, 0))
    o_ref[...] = jnp.concatenate(
        [z[0:DIM], z[DIM:2 * DIM], z[2 * DIM:3 * DIM], z[3 * DIM:4 * DIM]],
        axis=1)


def _back_transpose(g_flat):
    return pl.pallas_call(
        _back_body,
        grid=(OGRID,),
        in_specs=[pl.BlockSpec((OQ, 128), lambda t: (t, 0))],
        out_specs=pl.BlockSpec((DIM, OCOLS), lambda t: (0, t)),
        out_shape=jax.ShapeDtypeStruct((DIM, N_MINOR), jnp.float32),
    )(g_flat)


EA_CH = 6400
EA_NCH = 250


def _ea_copy_body(src_hbm, dst_hbm, buf_v):
    wid = lax.axis_index("s") * 2 + lax.axis_index("c")
    for k in range(8):
        cid = wid * 8 + k

        @pl.when(cid < EA_NCH)
        def _():
            off = cid * EA_CH
            pltpu.sync_copy(src_hbm.at[:, pl.ds(off, EA_CH)], buf_v)
            pltpu.sync_copy(buf_v, dst_hbm.at[:, pl.ds(off, EA_CH)])


def _sc_ea_copy(ea_t):
    mesh = plsc.VectorSubcoreMesh(core_axis_name="c", subcore_axis_name="s")
    f = pl.kernel(
        _ea_copy_body,
        out_type=jax.ShapeDtypeStruct(ea_t.shape, ea_t.dtype),
        mesh=mesh,
        scratch_types=[pltpu.VMEM((16, EA_CH), jnp.float32)],
        compiler_params=pltpu.CompilerParams(use_tc_tiling_on_sc=True),
    )
    return f(ea_t)


N_PAD_X = 102400
XSL = 3 * OCOLS


def _gather_body(table_hbm, x_hbm, out_hbm, xv, idx_v, rows_v, sem):
    wid = lax.axis_index("s") * 2 + lax.axis_index("c")
    base = wid * B_PER_W
    t0 = base // OCOLS
    pltpu.sync_copy(x_hbm.at[pl.ds(t0 * OCOLS, XSL)], xv)
    lane = lax.iota(jnp.int32, 16)
    for i in range(B_PER_W // 16):
        g = base + 16 * i + lane
        r = g & (OCOLS - 1)
        j_local = (g - r) - t0 * OCOLS + ((r & 3) << 9) + (r >> 2)
        xval = plsc.load_gather(xv, [j_local])
        u = xval & (TCOLS - 1)
        gv = (xval - u) + ((u & (QUART - 1)) << 2) + (u >> QUART_LOG2)
        idx_v[pl.ds(16 * i, 16)] = gv
    pltpu.async_copy(table_hbm.at[idx_v], rows_v, sem).wait()
    pltpu.sync_copy(rows_v, out_hbm.at[pl.ds(base, B_PER_W)])


def _gather(table_rows, x_pad):
    mesh = plsc.VectorSubcoreMesh(core_axis_name="c", subcore_axis_name="s")
    f = pl.kernel(
        _gather_body,
        out_type=jax.ShapeDtypeStruct((N_PAD, DIM), jnp.float32),
        mesh=mesh,
        scratch_types=[
            pltpu.VMEM((XSL,), jnp.int32),
            pltpu.VMEM((B_PER_W,), jnp.int32),
            pltpu.VMEM((B_PER_W, DIM), jnp.float32),
            pltpu.SemaphoreType.DMA,
        ],
        compiler_params=pltpu.CompilerParams(use_tc_tiling_on_sc=False,
                                             needs_layout_passes=False),
    )
    return f(table_rows, x_pad)


def kernel(x, edge_index, edge_attr, batch, depth, ptr, table):
    ea_out_t = _sc_ea_copy(edge_attr.T)
    table_t = table.T
    scratch = _detile(table_t)
    table_rows = scratch.reshape(ROWS_PAD, DIM)
    x_pad = jnp.pad(x.reshape(-1), (0, N_PAD_X - N))
    x_pad, ea_out_t = lax.optimization_barrier((x_pad, ea_out_t))
    g_rows = _gather(table_rows, x_pad)
    out_t = _back_transpose(g_rows.reshape(N_PAD * DIM // 128, 128))
    return (out_t.T[:N], edge_index, ea_out_t.T, batch, depth, ptr)


# speedup vs baseline: 13.4277x; 1.0959x over previous
import jax
import jax.numpy as jnp
from jax import lax
from jax.experimental import pallas as pl
from jax.experimental.pallas import tpu as pltpu
from jax.experimental.pallas import tpu_sc as plsc

N = 100000
DIM = 32
ROWS_PAD = 1015808
NW = 32

TCOLS = 16384
QUART = TCOLS // 4
QUART_LOG2 = QUART.bit_length() - 1
OUT_BLK = TCOLS * DIM // 128
SCRATCH_ROWS = ROWS_PAD * DIM // 128
GRID = -(-ROWS_PAD // TCOLS)

OCOLS = 2048
OQ = OCOLS // 4
N_PAD = 100352
B_PER_W = N_PAD // NW
OGRID = N_PAD // OCOLS
N_MINOR = 100096


def _transpose_body(x_ref, y_ref):
    x = x_ref[...]
    x4 = jnp.concatenate(
        [x[:, 0:QUART], x[:, QUART:2 * QUART], x[:, 2 * QUART:3 * QUART],
         x[:, 3 * QUART:4 * QUART]], axis=0)
    y_ref[...] = jnp.transpose(x4, (1, 0))


def _detile(table_t):
    return pl.pallas_call(
        _transpose_body,
        grid=(GRID,),
        in_specs=[pl.BlockSpec((DIM, TCOLS), lambda t: (0, t))],
        out_specs=pl.BlockSpec((OUT_BLK, 128), lambda t: (t, 0)),
        out_shape=jax.ShapeDtypeStruct((SCRATCH_ROWS, 128), jnp.float32),
    )(table_t)


def _back_body(g_ref, o_ref):
    z = jnp.transpose(g_ref[...], (1, 0))
    o_ref[...] = jnp.concatenate(
        [z[0:DIM], z[DIM:2 * DIM], z[2 * DIM:3 * DIM], z[3 * DIM:4 * DIM]],
        axis=1)


def _back_transpose(g_flat):
    return pl.pallas_call(
        _back_body,
        grid=(OGRID,),
        in_specs=[pl.BlockSpec((OQ, 128), lambda t: (t, 0))],
        out_specs=pl.BlockSpec((DIM, OCOLS), lambda t: (0, t)),
        out_shape=jax.ShapeDtypeStruct((DIM, N_MINOR), jnp.float32),
    )(g_flat)


EA_CH = 6400
EA_NCH = 250


def _ea_copy_body(src_hbm, dst_hbm, buf_v):
    wid = lax.axis_index("s") * 2 + lax.axis_index("c")
    for k in range(8):
        cid = wid * 8 + k

        @pl.when(cid < EA_NCH)
        def _():
            off = cid * EA_CH
            pltpu.sync_copy(src_hbm.at[:, pl.ds(off, EA_CH)], buf_v)
            pltpu.sync_copy(buf_v, dst_hbm.at[:, pl.ds(off, EA_CH)])


def _sc_ea_copy(ea_t):
    mesh = plsc.VectorSubcoreMesh(core_axis_name="c", subcore_axis_name="s")
    f = pl.kernel(
        _ea_copy_body,
        out_type=jax.ShapeDtypeStruct(ea_t.shape, ea_t.dtype),
        mesh=mesh,
        scratch_types=[pltpu.VMEM((16, EA_CH), jnp.float32)],
        compiler_params=pltpu.CompilerParams(use_tc_tiling_on_sc=True),
    )
    return f(ea_t)


N_PAD_X = 102400
XSL = 3 * OCOLS


def _gather_body(table_hbm, x_hbm, out_hbm, xv, idx_v, rows_v, sem):
    wid = lax.axis_index("s") * 2 + lax.axis_index("c")
    base = wid * B_PER_W
    t0 = base // OCOLS
    pltpu.sync_copy(x_hbm.at[pl.ds(t0 * OCOLS, XSL)], xv)
    lane = lax.iota(jnp.int32, 16)
    for i in range(B_PER_W // 16):
        g = base + 16 * i + lane
        r = g & (OCOLS - 1)
        j_local = (g - r) - t0 * OCOLS + ((r & 3) << 9) + (r >> 2)
        xval = plsc.load_gather(xv, [j_local])
        u = xval & (TCOLS - 1)
        gv = (xval - u) + ((u & (QUART - 1)) << 2) + (u >> QUART_LOG2)
        idx_v[pl.ds(16 * i, 16)] = gv
    pltpu.async_copy(table_hbm.at[idx_v], rows_v, sem).wait()
    pltpu.sync_copy(rows_v, out_hbm.at[pl.ds(base, B_PER_W)])


def _gather(table_rows, x_pad):
    mesh = plsc.VectorSubcoreMesh(core_axis_name="c", subcore_axis_name="s")
    f = pl.kernel(
        _gather_body,
        out_type=jax.ShapeDtypeStruct((N_PAD, DIM), jnp.float32),
        mesh=mesh,
        scratch_types=[
            pltpu.VMEM((XSL,), jnp.int32),
            pltpu.VMEM((B_PER_W,), jnp.int32),
            pltpu.VMEM((B_PER_W, DIM), jnp.float32),
            pltpu.SemaphoreType.DMA,
        ],
        compiler_params=pltpu.CompilerParams(use_tc_tiling_on_sc=False,
                                             needs_layout_passes=False),
    )
    return f(table_rows, x_pad)


def kernel(x, edge_index, edge_attr, batch, depth, ptr, table):
    ea_out_t = _sc_ea_copy(edge_attr.T)
    table_t = table.T
    scratch = _detile(table_t)
    table_rows = scratch.reshape(ROWS_PAD, DIM)
    x_pad = jnp.pad(x.reshape(-1), (0, N_PAD_X - N))
    x_pad, ea_out_t = lax.optimization_barrier((x_pad, ea_out_t))
    g_rows = _gather(table_rows, x_pad)
    out_t = _back_transpose(g_rows.reshape(N_PAD * DIM // 128, 128))
    return (out_t.T[:N], edge_index, ea_out_t.T, batch, depth, ptr)


# speedup vs baseline: 13.7047x; 1.0206x over previous
import jax
import jax.numpy as jnp
from jax import lax
from jax.experimental import pallas as pl
from jax.experimental.pallas import tpu as pltpu
from jax.experimental.pallas import tpu_sc as plsc

N = 100000
DIM = 32
ROWS_PAD = 1015808
NW = 32

TCOLS = 32768
QUART = TCOLS // 4
QUART_LOG2 = QUART.bit_length() - 1
OUT_BLK = TCOLS * DIM // 128
SCRATCH_ROWS = ROWS_PAD * DIM // 128
GRID = -(-ROWS_PAD // TCOLS)

OCOLS = 2048
OQ = OCOLS // 4
N_PAD = 100352
B_PER_W = N_PAD // NW
OGRID = N_PAD // OCOLS
N_MINOR = 100096


def _transpose_body(x_ref, y_ref):
    x = x_ref[...]
    x4 = jnp.concatenate(
        [x[:, 0:QUART], x[:, QUART:2 * QUART], x[:, 2 * QUART:3 * QUART],
         x[:, 3 * QUART:4 * QUART]], axis=0)
    y_ref[...] = jnp.transpose(x4, (1, 0))


def _detile(table_t):
    return pl.pallas_call(
        _transpose_body,
        grid=(GRID,),
        in_specs=[pl.BlockSpec((DIM, TCOLS), lambda t: (0, t))],
        out_specs=pl.BlockSpec((OUT_BLK, 128), lambda t: (t, 0)),
        out_shape=jax.ShapeDtypeStruct((SCRATCH_ROWS, 128), jnp.float32),
    )(table_t)


def _back_body(g_ref, o_ref):
    z = jnp.transpose(g_ref[...], (1, 0))
    o_ref[...] = jnp.concatenate(
        [z[0:DIM], z[DIM:2 * DIM], z[2 * DIM:3 * DIM], z[3 * DIM:4 * DIM]],
        axis=1)


def _back_transpose(g_flat):
    return pl.pallas_call(
        _back_body,
        grid=(OGRID,),
        in_specs=[pl.BlockSpec((OQ, 128), lambda t: (t, 0))],
        out_specs=pl.BlockSpec((DIM, OCOLS), lambda t: (0, t)),
        out_shape=jax.ShapeDtypeStruct((DIM, N_MINOR), jnp.float32),
    )(g_flat)


EA_CH = 6400
EA_NCH = 250


def _ea_copy_body(src_hbm, dst_hbm, buf_v):
    wid = lax.axis_index("s") * 2 + lax.axis_index("c")
    for k in range(8):
        cid = wid * 8 + k

        @pl.when(cid < EA_NCH)
        def _():
            off = cid * EA_CH
            pltpu.sync_copy(src_hbm.at[:, pl.ds(off, EA_CH)], buf_v)
            pltpu.sync_copy(buf_v, dst_hbm.at[:, pl.ds(off, EA_CH)])


def _sc_ea_copy(ea_t):
    mesh = plsc.VectorSubcoreMesh(core_axis_name="c", subcore_axis_name="s")
    f = pl.kernel(
        _ea_copy_body,
        out_type=jax.ShapeDtypeStruct(ea_t.shape, ea_t.dtype),
        mesh=mesh,
        scratch_types=[pltpu.VMEM((16, EA_CH), jnp.float32)],
        compiler_params=pltpu.CompilerParams(use_tc_tiling_on_sc=True),
    )
    return f(ea_t)


N_PAD_X = 102400
XSL = 3 * OCOLS


def _gather_body(table_hbm, x_hbm, out_hbm, xv, idx_v, rows_v, sem):
    wid = lax.axis_index("s") * 2 + lax.axis_index("c")
    base = wid * B_PER_W
    t0 = base // OCOLS
    pltpu.sync_copy(x_hbm.at[pl.ds(t0 * OCOLS, XSL)], xv)
    lane = lax.iota(jnp.int32, 16)
    for i in range(B_PER_W // 16):
        g = base + 16 * i + lane
        r = g & (OCOLS - 1)
        j_local = (g - r) - t0 * OCOLS + ((r & 3) << 9) + (r >> 2)
        xval = plsc.load_gather(xv, [j_local])
        u = xval & (TCOLS - 1)
        gv = (xval - u) + ((u & (QUART - 1)) << 2) + (u >> QUART_LOG2)
        idx_v[pl.ds(16 * i, 16)] = gv
    pltpu.async_copy(table_hbm.at[idx_v], rows_v, sem).wait()
    pltpu.sync_copy(rows_v, out_hbm.at[pl.ds(base, B_PER_W)])


def _gather(table_rows, x_pad):
    mesh = plsc.VectorSubcoreMesh(core_axis_name="c", subcore_axis_name="s")
    f = pl.kernel(
        _gather_body,
        out_type=jax.ShapeDtypeStruct((N_PAD, DIM), jnp.float32),
        mesh=mesh,
        scratch_types=[
            pltpu.VMEM((XSL,), jnp.int32),
            pltpu.VMEM((B_PER_W,), jnp.int32),
            pltpu.VMEM((B_PER_W, DIM), jnp.float32),
            pltpu.SemaphoreType.DMA,
        ],
        compiler_params=pltpu.CompilerParams(use_tc_tiling_on_sc=False,
                                             needs_layout_passes=False),
    )
    return f(table_rows, x_pad)


def kernel(x, edge_index, edge_attr, batch, depth, ptr, table):
    ea_out_t = _sc_ea_copy(edge_attr.T)
    table_t = table.T
    scratch = _detile(table_t)
    table_rows = scratch.reshape(ROWS_PAD, DIM)
    x_pad = jnp.pad(x.reshape(-1), (0, N_PAD_X - N))
    x_pad, ea_out_t = lax.optimization_barrier((x_pad, ea_out_t))
    g_rows = _gather(table_rows, x_pad)
    out_t = _back_transpose(g_rows.reshape(N_PAD * DIM // 128, 128))
    return (out_t.T[:N], edge_index, ea_out_t.T, batch, depth, ptr)
